# Initial kernel scaffold; baseline (speedup 1.0000x reference)
import jax
import jax.numpy as jnp
from jax.experimental import pallas as pl


def kernel(support_data, query_data, W, labels):
    raise NotImplementedError("write your pallas kernel here")


# speedup vs baseline: 2.1523x; 2.1523x over previous
import functools

import jax
import jax.numpy as jnp
from jax import lax
from jax.experimental import pallas as pl
from jax.experimental.pallas import tpu as pltpu
from jax.experimental.pallas import tpu_sc as plsc

NUM_CLASSES = 100
N_SUPPORT = 100000
Q = 4096
D = 128

TILE = 80
NTILES = N_SUPPORT // TILE
NW = 32

EMB_TILE = 1000


def _embed_body(x_ref, w_ref, o_ref):
    o_ref[...] = jnp.dot(x_ref[...], w_ref[...].T)


def _tc_embed(support_data, W):
    return pl.pallas_call(
        _embed_body,
        grid=(N_SUPPORT // EMB_TILE,),
        in_specs=[pl.BlockSpec((EMB_TILE, D), lambda i: (i, 0)),
                  pl.BlockSpec((D, D), lambda i: (0, 0))],
        out_specs=pl.BlockSpec((EMB_TILE, D), lambda i: (i, 0)),
        out_shape=jax.ShapeDtypeStruct((N_SUPPORT, D), jnp.float32),
    )(support_data, W)


def _sc_segment_body(sup_hbm, lab_hbm, sums_hbm, cnts_hbm, buf, labv, acc, cnt):
    c = lax.axis_index("c")
    s = lax.axis_index("s")
    wid = s * 2 + c
    ntiles = NTILES // NW + jnp.where(wid < NTILES % NW, 1, 0)

    iota = lax.iota(jnp.int32, 16)
    zero16 = jnp.zeros((16,), jnp.float32)
    one0 = jnp.where(iota == 0, 1.0, 0.0).astype(jnp.float32)

    def _zero_row(i, carry):
        for k in range(8):
            acc[pl.ds(i * D + k * 16, 16)] = zero16
        cnt[pl.ds(i * 16, 16)] = zero16
        return carry

    lax.fori_loop(0, NUM_CLASSES, _zero_row, 0)

    def _tile(t, carry):
        tile = wid + NW * t
        row0 = tile * TILE
        pltpu.sync_copy(sup_hbm.at[pl.ds(row0, TILE), :], buf)
        pltpu.sync_copy(lab_hbm.at[pl.ds(row0, TILE)], labv)

        def _group(g, rc):
            lv = labv[pl.ds(g * 16, 16)]
            for j in range(16):
                r = g * 16 + j
                base = lv[j] * D
                for k in range(8):
                    v = buf[r, pl.ds(k * 16, 16)]
                    plsc.addupdate_scatter(acc, [iota + (base + k * 16)], v)
                plsc.addupdate_scatter(cnt, [iota + lv[j] * 16], one0)
            return rc

        lax.fori_loop(0, TILE // 16, _group, 0)
        return carry

    lax.fori_loop(0, ntiles, _tile, 0)

    pltpu.sync_copy(acc, sums_hbm.at[wid])
    pltpu.sync_copy(cnt, cnts_hbm.at[wid])


@functools.partial(
    pl.kernel,
    out_type=(
        jax.ShapeDtypeStruct((NW, NUM_CLASSES * D), jnp.float32),
        jax.ShapeDtypeStruct((NW, NUM_CLASSES * 16), jnp.float32),
    ),
    mesh=plsc.VectorSubcoreMesh(core_axis_name="c", subcore_axis_name="s"),
    compiler_params=pltpu.CompilerParams(needs_layout_passes=False),
    scratch_types=(
        pltpu.VMEM((TILE, D), jnp.float32),
        pltpu.VMEM((TILE,), jnp.int32),
        pltpu.VMEM((NUM_CLASSES * D,), jnp.float32),
        pltpu.VMEM((NUM_CLASSES * 16,), jnp.float32),
    ),
)
def _sc_segment_sum(sup, lab, sums, cnts, buf, labv, acc, cnt):
    _sc_segment_body(sup, lab, sums, cnts, buf, labv, acc, cnt)


def _tc_tail_body(sums_ref, cnts_ref, q_ref, w_ref, pred_ref, probs_ref):
    total = jnp.sum(sums_ref[...], axis=0).reshape(NUM_CLASSES, D)
    counts = jnp.sum(cnts_ref[...], axis=0)[:, 0:1]
    protos = total / jnp.maximum(counts, 1.0)
    qe = jnp.dot(q_ref[...], w_ref[...].T)
    pn = jnp.sum(protos * protos, axis=1)[None, :]
    qp = lax.dot_general(qe, protos, (((1,), (1,)), ((), ())),
                         precision=lax.Precision.HIGHEST,
                         preferred_element_type=jnp.float32)
    scores = 2.0 * qp - pn
    m = jnp.max(scores, axis=1, keepdims=True)
    e = jnp.exp(scores - m)
    probs = e / jnp.sum(e, axis=1, keepdims=True)
    pred_ref[...] = jnp.argmax(scores, axis=1, keepdims=True).astype(jnp.int32)
    probs_ref[...] = probs


def _tc_tail(sums_p, cnts_p, query_data, W):
    return pl.pallas_call(
        _tc_tail_body,
        out_shape=(
            jax.ShapeDtypeStruct((Q, 1), jnp.int32),
            jax.ShapeDtypeStruct((Q, NUM_CLASSES), jnp.float32),
        ),
    )(sums_p, cnts_p.reshape(NW, NUM_CLASSES, 16), query_data, W)


def kernel(support_data, query_data, W, labels):
    lab32 = labels.astype(jnp.int32)
    emb = _tc_embed(support_data, W)
    sums_p, cnts_p = _sc_segment_sum(emb, lab32)
    pred, probs = _tc_tail(sums_p, cnts_p, query_data, W)
    return pred.reshape((Q,)), probs


# speedup vs baseline: 3.2866x; 1.5270x over previous
import functools

import jax
import jax.numpy as jnp
from jax import lax
from jax.experimental import pallas as pl
from jax.experimental.pallas import tpu as pltpu
from jax.experimental.pallas import tpu_sc as plsc

NUM_CLASSES = 100
N_SUPPORT = 100000
Q = 4096
D = 128

TILE = 400
NW = 32

NCHUNK = 1
CHUNK = N_SUPPORT // NCHUNK
CTILES = CHUNK // TILE

EMB_TILE = 1000


def _embed_body(x_ref, w_ref, o_ref):
    o_ref[...] = jnp.dot(x_ref[...], w_ref[...].T)


def _tc_embed(support_data, W):
    n = support_data.shape[0]
    return pl.pallas_call(
        _embed_body,
        grid=(n // EMB_TILE,),
        in_specs=[pl.BlockSpec((EMB_TILE, D), lambda i: (i, 0)),
                  pl.BlockSpec((D, D), lambda i: (0, 0))],
        out_specs=pl.BlockSpec((EMB_TILE, D), lambda i: (i, 0)),
        out_shape=jax.ShapeDtypeStruct((n, D), jnp.float32),
    )(support_data, W)


def _sc_segment_body(sup_hbm, lab_hbm, sums_hbm, cnts_hbm, buf, labv, acc, cnt):
    c = lax.axis_index("c")
    s = lax.axis_index("s")
    wid = s * 2 + c
    ntiles = CTILES // NW + jnp.where(wid < CTILES % NW, 1, 0)

    iota = lax.iota(jnp.int32, 16)
    zero16 = jnp.zeros((16,), jnp.float32)
    one0 = jnp.where(iota == 0, 1.0, 0.0).astype(jnp.float32)
    sixteen0 = jnp.where(iota == 0, 16.0, 0.0).astype(jnp.float32)

    def _zero_row(i, carry):
        for k in range(8):
            acc[pl.ds(i * D + k * 16, 16)] = zero16
        cnt[pl.ds(i * 16, 16)] = zero16
        return carry

    lax.fori_loop(0, NUM_CLASSES, _zero_row, 0)

    def _tile(t, carry):
        tile = wid + NW * t
        row0 = tile * TILE
        pltpu.sync_copy(sup_hbm.at[pl.ds(row0, TILE), :], buf)
        pltpu.sync_copy(lab_hbm.at[pl.ds(row0, TILE)], labv)

        def _group(g, rc):
            lv = labv[pl.ds(g * 16, 16)]
            r0 = g * 16
            l0 = lv[0]
            l15 = lv[15]

            @pl.when(l0 == l15)
            def _fast():
                base = l0 * D
                for k in range(8):
                    vs = [buf[r0 + j, pl.ds(k * 16, 16)] for j in range(16)]
                    while len(vs) > 1:
                        vs = [vs[i] + vs[i + 1] for i in range(0, len(vs), 2)]
                    plsc.addupdate_scatter(acc, [iota + (base + k * 16)], vs[0])
                plsc.addupdate_scatter(cnt, [iota + l0 * 16], sixteen0)

            @pl.when(l0 != l15)
            def _slow():
                for j in range(16):
                    base = lv[j] * D
                    for k in range(8):
                        v = buf[r0 + j, pl.ds(k * 16, 16)]
                        plsc.addupdate_scatter(acc, [iota + (base + k * 16)], v)
                    plsc.addupdate_scatter(cnt, [iota + lv[j] * 16], one0)

            return rc

        lax.fori_loop(0, TILE // 16, _group, 0)
        return carry

    lax.fori_loop(0, ntiles, _tile, 0)

    pltpu.sync_copy(acc, sums_hbm.at[wid])
    pltpu.sync_copy(cnt, cnts_hbm.at[wid])


@functools.partial(
    pl.kernel,
    out_type=(
        jax.ShapeDtypeStruct((NW, NUM_CLASSES * D), jnp.float32),
        jax.ShapeDtypeStruct((NW, NUM_CLASSES * 16), jnp.float32),
    ),
    mesh=plsc.VectorSubcoreMesh(core_axis_name="c", subcore_axis_name="s"),
    compiler_params=pltpu.CompilerParams(needs_layout_passes=False),
    scratch_types=(
        pltpu.VMEM((TILE, D), jnp.float32),
        pltpu.VMEM((TILE,), jnp.int32),
        pltpu.VMEM((NUM_CLASSES * D,), jnp.float32),
        pltpu.VMEM((NUM_CLASSES * 16,), jnp.float32),
    ),
)
def _sc_segment_sum(sup, lab, sums, cnts, buf, labv, acc, cnt):
    _sc_segment_body(sup, lab, sums, cnts, buf, labv, acc, cnt)


def _tc_tail_body(sums_ref, cnts_ref, q_ref, w_ref, pred_ref, probs_ref):
    total = jnp.sum(sums_ref[...], axis=0).reshape(NUM_CLASSES, D)
    counts = jnp.sum(cnts_ref[...], axis=0)[:, 0:1]
    protos = total / jnp.maximum(counts, 1.0)
    qe = jnp.dot(q_ref[...], w_ref[...].T)
    pn = jnp.sum(protos * protos, axis=1)[None, :]
    qp = lax.dot_general(qe, protos, (((1,), (1,)), ((), ())),
                         precision=lax.Precision.HIGHEST,
                         preferred_element_type=jnp.float32)
    scores = 2.0 * qp - pn
    m = jnp.max(scores, axis=1, keepdims=True)
    e = jnp.exp(scores - m)
    probs = e / jnp.sum(e, axis=1, keepdims=True)
    pred_ref[...] = jnp.argmax(scores, axis=1, keepdims=True).astype(jnp.int32)
    probs_ref[...] = probs


def _tc_tail(sums_p, cnts_p, query_data, W):
    nw = sums_p.shape[0]
    return pl.pallas_call(
        _tc_tail_body,
        out_shape=(
            jax.ShapeDtypeStruct((Q, 1), jnp.int32),
            jax.ShapeDtypeStruct((Q, NUM_CLASSES), jnp.float32),
        ),
    )(sums_p, cnts_p.reshape(nw, NUM_CLASSES, 16), query_data, W)


def kernel(support_data, query_data, W, labels):
    lab32 = labels.astype(jnp.int32)
    sums_parts, cnts_parts = [], []
    for k in range(NCHUNK):
        emb = _tc_embed(support_data[k * CHUNK:(k + 1) * CHUNK], W)
        s_p, c_p = _sc_segment_sum(emb, lab32[k * CHUNK:(k + 1) * CHUNK])
        sums_parts.append(s_p)
        cnts_parts.append(c_p)
    sums_p = jnp.concatenate(sums_parts, axis=0)
    cnts_p = jnp.concatenate(cnts_parts, axis=0)
    pred, probs = _tc_tail(sums_p, cnts_p, query_data, W)
    return pred.reshape((Q,)), probs


# speedup vs baseline: 3.8923x; 1.1843x over previous
import functools

import jax
import jax.numpy as jnp
from jax import lax
from jax.experimental import pallas as pl
from jax.experimental.pallas import tpu as pltpu
from jax.experimental.pallas import tpu_sc as plsc

NUM_CLASSES = 100
N_SUPPORT = 100000
Q = 4096
D = 128

TILE = 400
NW = 32

NCHUNK = 1
CHUNK = N_SUPPORT // NCHUNK
CTILES = CHUNK // TILE

EMB_TILE = 2000


def _embed_body(x_ref, w_ref, o_ref):
    o_ref[...] = jnp.dot(x_ref[...], w_ref[...].T)


def _tc_embed(support_data, W):
    n = support_data.shape[0]
    return pl.pallas_call(
        _embed_body,
        grid=(n // EMB_TILE,),
        in_specs=[pl.BlockSpec((EMB_TILE, D), lambda i: (i, 0)),
                  pl.BlockSpec((D, D), lambda i: (0, 0))],
        out_specs=pl.BlockSpec((EMB_TILE, D), lambda i: (i, 0)),
        out_shape=jax.ShapeDtypeStruct((n, D), jnp.float32),
    )(support_data, W)


def _sc_segment_body(sup_hbm, lab_hbm, sums_hbm, cnts_hbm, buf, labv, acc, cnt):
    c = lax.axis_index("c")
    s = lax.axis_index("s")
    wid = s * 2 + c
    ntiles = CTILES // NW + jnp.where(wid < CTILES % NW, 1, 0)

    iota = lax.iota(jnp.int32, 16)
    zero16 = jnp.zeros((16,), jnp.float32)
    one0 = jnp.where(iota == 0, 1.0, 0.0).astype(jnp.float32)
    sixteen0 = jnp.where(iota == 0, 16.0, 0.0).astype(jnp.float32)

    def _zero_row(i, carry):
        for k in range(8):
            acc[pl.ds(i * D + k * 16, 16)] = zero16
        cnt[pl.ds(i * 16, 16)] = zero16
        return carry

    lax.fori_loop(0, NUM_CLASSES, _zero_row, 0)

    def _tile(t, carry):
        tile = wid + NW * t
        row0 = tile * TILE
        pltpu.sync_copy(sup_hbm.at[pl.ds(row0, TILE), :], buf)
        pltpu.sync_copy(lab_hbm.at[pl.ds(row0, TILE)], labv)

        def _group(g, rc):
            lv = labv[pl.ds(g * 16, 16)]
            r0 = g * 16
            l0 = lv[0]
            l15 = lv[15]

            @pl.when(l0 == l15)
            def _fast():
                base = l0 * D
                for k in range(8):
                    vs = [buf[r0 + j, pl.ds(k * 16, 16)] for j in range(16)]
                    while len(vs) > 1:
                        vs = [vs[i] + vs[i + 1] for i in range(0, len(vs), 2)]
                    plsc.addupdate_scatter(acc, [iota + (base + k * 16)], vs[0])
                plsc.addupdate_scatter(cnt, [iota + l0 * 16], sixteen0)

            @pl.when(l0 != l15)
            def _slow():
                for j in range(16):
                    base = lv[j] * D
                    for k in range(8):
                        v = buf[r0 + j, pl.ds(k * 16, 16)]
                        plsc.addupdate_scatter(acc, [iota + (base + k * 16)], v)
                    plsc.addupdate_scatter(cnt, [iota + lv[j] * 16], one0)

            return rc

        lax.fori_loop(0, TILE // 16, _group, 0)
        return carry

    lax.fori_loop(0, ntiles, _tile, 0)

    pltpu.sync_copy(acc, sums_hbm.at[wid])
    pltpu.sync_copy(cnt, cnts_hbm.at[wid])


@functools.partial(
    pl.kernel,
    out_type=(
        jax.ShapeDtypeStruct((NW, NUM_CLASSES * D), jnp.float32),
        jax.ShapeDtypeStruct((NW, NUM_CLASSES * 16), jnp.float32),
    ),
    mesh=plsc.VectorSubcoreMesh(core_axis_name="c", subcore_axis_name="s"),
    compiler_params=pltpu.CompilerParams(needs_layout_passes=False),
    scratch_types=(
        pltpu.VMEM((TILE, D), jnp.float32),
        pltpu.VMEM((TILE,), jnp.int32),
        pltpu.VMEM((NUM_CLASSES * D,), jnp.float32),
        pltpu.VMEM((NUM_CLASSES * 16,), jnp.float32),
    ),
)
def _sc_segment_sum(sup, lab, sums, cnts, buf, labv, acc, cnt):
    _sc_segment_body(sup, lab, sums, cnts, buf, labv, acc, cnt)


def _tc_tail_body(sums_ref, cnts_ref, q_ref, w_ref, pred_ref, probs_ref):
    total = jnp.sum(sums_ref[...], axis=0).reshape(NUM_CLASSES, D)
    counts = jnp.sum(cnts_ref[...], axis=0)[:, 0:1]
    protos = total / jnp.maximum(counts, 1.0)
    qe = jnp.dot(q_ref[...], w_ref[...].T)
    pn = jnp.sum(protos * protos, axis=1)[None, :]
    qp = lax.dot_general(qe, protos, (((1,), (1,)), ((), ())),
                         precision=lax.Precision.HIGHEST,
                         preferred_element_type=jnp.float32)
    scores = 2.0 * qp - pn
    m = jnp.max(scores, axis=1, keepdims=True)
    e = jnp.exp(scores - m)
    probs = e / jnp.sum(e, axis=1, keepdims=True)
    pred_ref[...] = jnp.argmax(scores, axis=1, keepdims=True).astype(jnp.int32)
    probs_ref[...] = probs


def _tc_tail(sums_p, cnts_p, query_data, W):
    nw = sums_p.shape[0]
    return pl.pallas_call(
        _tc_tail_body,
        out_shape=(
            jax.ShapeDtypeStruct((Q, 1), jnp.int32),
            jax.ShapeDtypeStruct((Q, NUM_CLASSES), jnp.float32),
        ),
    )(sums_p, cnts_p.reshape(nw, NUM_CLASSES, 16), query_data, W)


def kernel(support_data, query_data, W, labels):
    lab32 = labels.astype(jnp.int32)
    sums_parts, cnts_parts = [], []
    for k in range(NCHUNK):
        emb = _tc_embed(support_data[k * CHUNK:(k + 1) * CHUNK], W)
        s_p, c_p = _sc_segment_sum(emb, lab32[k * CHUNK:(k + 1) * CHUNK])
        sums_parts.append(s_p)
        cnts_parts.append(c_p)
    sums_p = jnp.concatenate(sums_parts, axis=0)
    cnts_p = jnp.concatenate(cnts_parts, axis=0)
    pred, probs = _tc_tail(sums_p, cnts_p, query_data, W)
    return pred.reshape((Q,)), probs


# speedup vs baseline: 4.3895x; 1.1277x over previous
import functools

import jax
import jax.numpy as jnp
from jax import lax
from jax.experimental import pallas as pl
from jax.experimental.pallas import tpu as pltpu
from jax.experimental.pallas import tpu_sc as plsc

NUM_CLASSES = 100
N_SUPPORT = 100000
Q = 4096
D = 128

TILE = 400
NW = 32

NCHUNK = 1
CHUNK = N_SUPPORT // NCHUNK
CTILES = CHUNK // TILE

EMB_TILE = 5000


def _embed_body(x_ref, w_ref, o_ref):
    o_ref[...] = jnp.dot(x_ref[...], w_ref[...].T)


def _tc_embed(support_data, W):
    n = support_data.shape[0]
    return pl.pallas_call(
        _embed_body,
        grid=(n // EMB_TILE,),
        in_specs=[pl.BlockSpec((EMB_TILE, D), lambda i: (i, 0)),
                  pl.BlockSpec((D, D), lambda i: (0, 0))],
        out_specs=pl.BlockSpec((EMB_TILE, D), lambda i: (i, 0)),
        out_shape=jax.ShapeDtypeStruct((n, D), jnp.float32),
    )(support_data, W)


def _sc_segment_body(sup_hbm, lab_hbm, sums_hbm, cnts_hbm, buf, labv, acc, cnt):
    c = lax.axis_index("c")
    s = lax.axis_index("s")
    wid = s * 2 + c
    ntiles = CTILES // NW + jnp.where(wid < CTILES % NW, 1, 0)

    iota = lax.iota(jnp.int32, 16)
    zero16 = jnp.zeros((16,), jnp.float32)
    one0 = jnp.where(iota == 0, 1.0, 0.0).astype(jnp.float32)
    sixteen0 = jnp.where(iota == 0, 16.0, 0.0).astype(jnp.float32)

    def _zero_row(i, carry):
        for k in range(8):
            acc[pl.ds(i * D + k * 16, 16)] = zero16
        cnt[pl.ds(i * 16, 16)] = zero16
        return carry

    lax.fori_loop(0, NUM_CLASSES, _zero_row, 0)

    def _tile(t, carry):
        tile = wid + NW * t
        row0 = tile * TILE
        pltpu.sync_copy(sup_hbm.at[pl.ds(row0, TILE), :], buf)
        pltpu.sync_copy(lab_hbm.at[pl.ds(row0, TILE)], labv)

        def _group(g, rc):
            lv = labv[pl.ds(g * 16, 16)]
            r0 = g * 16
            l0 = lv[0]
            l15 = lv[15]

            @pl.when(l0 == l15)
            def _fast():
                base = l0 * D
                for k in range(8):
                    vs = [buf[r0 + j, pl.ds(k * 16, 16)] for j in range(16)]
                    while len(vs) > 1:
                        vs = [vs[i] + vs[i + 1] for i in range(0, len(vs), 2)]
                    plsc.addupdate_scatter(acc, [iota + (base + k * 16)], vs[0])
                plsc.addupdate_scatter(cnt, [iota + l0 * 16], sixteen0)

            @pl.when(l0 != l15)
            def _slow():
                for j in range(16):
                    base = lv[j] * D
                    for k in range(8):
                        v = buf[r0 + j, pl.ds(k * 16, 16)]
                        plsc.addupdate_scatter(acc, [iota + (base + k * 16)], v)
                    plsc.addupdate_scatter(cnt, [iota + lv[j] * 16], one0)

            return rc

        lax.fori_loop(0, TILE // 16, _group, 0)
        return carry

    lax.fori_loop(0, ntiles, _tile, 0)

    pltpu.sync_copy(acc, sums_hbm.at[wid])
    pltpu.sync_copy(cnt, cnts_hbm.at[wid])


@functools.partial(
    pl.kernel,
    out_type=(
        jax.ShapeDtypeStruct((NW, NUM_CLASSES * D), jnp.float32),
        jax.ShapeDtypeStruct((NW, NUM_CLASSES * 16), jnp.float32),
    ),
    mesh=plsc.VectorSubcoreMesh(core_axis_name="c", subcore_axis_name="s"),
    compiler_params=pltpu.CompilerParams(needs_layout_passes=False),
    scratch_types=(
        pltpu.VMEM((TILE, D), jnp.float32),
        pltpu.VMEM((TILE,), jnp.int32),
        pltpu.VMEM((NUM_CLASSES * D,), jnp.float32),
        pltpu.VMEM((NUM_CLASSES * 16,), jnp.float32),
    ),
)
def _sc_segment_sum(sup, lab, sums, cnts, buf, labv, acc, cnt):
    _sc_segment_body(sup, lab, sums, cnts, buf, labv, acc, cnt)


def _tc_tail_body(sums_ref, cnts_ref, q_ref, w_ref, pred_ref, probs_ref):
    total = jnp.sum(sums_ref[...], axis=0).reshape(NUM_CLASSES, D)
    counts = jnp.sum(cnts_ref[...], axis=0)[:, 0:1]
    protos = total / jnp.maximum(counts, 1.0)
    qe = jnp.dot(q_ref[...], w_ref[...].T)
    pn = jnp.sum(protos * protos, axis=1)[None, :]
    qp = lax.dot_general(qe, protos, (((1,), (1,)), ((), ())),
                         precision=lax.Precision.HIGHEST,
                         preferred_element_type=jnp.float32)
    scores = 2.0 * qp - pn
    m = jnp.max(scores, axis=1, keepdims=True)
    e = jnp.exp(scores - m)
    probs = e / jnp.sum(e, axis=1, keepdims=True)
    pred_ref[...] = jnp.argmax(scores, axis=1, keepdims=True).astype(jnp.int32)
    probs_ref[...] = probs


def _tc_tail(sums_p, cnts_p, query_data, W):
    nw = sums_p.shape[0]
    return pl.pallas_call(
        _tc_tail_body,
        out_shape=(
            jax.ShapeDtypeStruct((Q, 1), jnp.int32),
            jax.ShapeDtypeStruct((Q, NUM_CLASSES), jnp.float32),
        ),
    )(sums_p, cnts_p.reshape(nw, NUM_CLASSES, 16), query_data, W)


def kernel(support_data, query_data, W, labels):
    lab32 = labels.astype(jnp.int32)
    sums_parts, cnts_parts = [], []
    for k in range(NCHUNK):
        emb = _tc_embed(support_data[k * CHUNK:(k + 1) * CHUNK], W)
        s_p, c_p = _sc_segment_sum(emb, lab32[k * CHUNK:(k + 1) * CHUNK])
        sums_parts.append(s_p)
        cnts_parts.append(c_p)
    sums_p = jnp.concatenate(sums_parts, axis=0)
    cnts_p = jnp.concatenate(cnts_parts, axis=0)
    pred, probs = _tc_tail(sums_p, cnts_p, query_data, W)
    return pred.reshape((Q,)), probs


# speedup vs baseline: 4.5944x; 1.0467x over previous
import functools

import jax
import jax.numpy as jnp
from jax import lax
from jax.experimental import pallas as pl
from jax.experimental.pallas import tpu as pltpu
from jax.experimental.pallas import tpu_sc as plsc

NUM_CLASSES = 100
N_SUPPORT = 100000
Q = 4096
D = 128

TILE = 400
NW = 32

NCHUNK = 1
CHUNK = N_SUPPORT // NCHUNK
CTILES = CHUNK // TILE

EMB_TILE = 10000


def _embed_body(x_ref, w_ref, o_ref):
    o_ref[...] = jnp.dot(x_ref[...], w_ref[...].T)


def _tc_embed(support_data, W):
    n = support_data.shape[0]
    return pl.pallas_call(
        _embed_body,
        grid=(n // EMB_TILE,),
        in_specs=[pl.BlockSpec((EMB_TILE, D), lambda i: (i, 0)),
                  pl.BlockSpec((D, D), lambda i: (0, 0))],
        out_specs=pl.BlockSpec((EMB_TILE, D), lambda i: (i, 0)),
        out_shape=jax.ShapeDtypeStruct((n, D), jnp.float32),
    )(support_data, W)


def _sc_segment_body(sup_hbm, lab_hbm, sums_hbm, cnts_hbm, buf, labv, acc, cnt):
    c = lax.axis_index("c")
    s = lax.axis_index("s")
    wid = s * 2 + c
    ntiles = CTILES // NW + jnp.where(wid < CTILES % NW, 1, 0)

    iota = lax.iota(jnp.int32, 16)
    zero16 = jnp.zeros((16,), jnp.float32)
    one0 = jnp.where(iota == 0, 1.0, 0.0).astype(jnp.float32)
    sixteen0 = jnp.where(iota == 0, 16.0, 0.0).astype(jnp.float32)

    def _zero_row(i, carry):
        for k in range(8):
            acc[pl.ds(i * D + k * 16, 16)] = zero16
        cnt[pl.ds(i * 16, 16)] = zero16
        return carry

    lax.fori_loop(0, NUM_CLASSES, _zero_row, 0)

    def _tile(t, carry):
        tile = wid + NW * t
        row0 = tile * TILE
        pltpu.sync_copy(sup_hbm.at[pl.ds(row0, TILE), :], buf)
        pltpu.sync_copy(lab_hbm.at[pl.ds(row0, TILE)], labv)

        def _group(g, rc):
            lv = labv[pl.ds(g * 16, 16)]
            r0 = g * 16
            l0 = lv[0]
            l15 = lv[15]

            @pl.when(l0 == l15)
            def _fast():
                base = l0 * D
                for k in range(8):
                    vs = [buf[r0 + j, pl.ds(k * 16, 16)] for j in range(16)]
                    while len(vs) > 1:
                        vs = [vs[i] + vs[i + 1] for i in range(0, len(vs), 2)]
                    plsc.addupdate_scatter(acc, [iota + (base + k * 16)], vs[0])
                plsc.addupdate_scatter(cnt, [iota + l0 * 16], sixteen0)

            @pl.when(l0 != l15)
            def _slow():
                for j in range(16):
                    base = lv[j] * D
                    for k in range(8):
                        v = buf[r0 + j, pl.ds(k * 16, 16)]
                        plsc.addupdate_scatter(acc, [iota + (base + k * 16)], v)
                    plsc.addupdate_scatter(cnt, [iota + lv[j] * 16], one0)

            return rc

        lax.fori_loop(0, TILE // 16, _group, 0)
        return carry

    lax.fori_loop(0, ntiles, _tile, 0)

    pltpu.sync_copy(acc, sums_hbm.at[wid])
    pltpu.sync_copy(cnt, cnts_hbm.at[wid])


@functools.partial(
    pl.kernel,
    out_type=(
        jax.ShapeDtypeStruct((NW, NUM_CLASSES * D), jnp.float32),
        jax.ShapeDtypeStruct((NW, NUM_CLASSES * 16), jnp.float32),
    ),
    mesh=plsc.VectorSubcoreMesh(core_axis_name="c", subcore_axis_name="s"),
    compiler_params=pltpu.CompilerParams(needs_layout_passes=False),
    scratch_types=(
        pltpu.VMEM((TILE, D), jnp.float32),
        pltpu.VMEM((TILE,), jnp.int32),
        pltpu.VMEM((NUM_CLASSES * D,), jnp.float32),
        pltpu.VMEM((NUM_CLASSES * 16,), jnp.float32),
    ),
)
def _sc_segment_sum(sup, lab, sums, cnts, buf, labv, acc, cnt):
    _sc_segment_body(sup, lab, sums, cnts, buf, labv, acc, cnt)


def _tc_tail_body(sums_ref, cnts_ref, q_ref, w_ref, pred_ref, probs_ref):
    total = jnp.sum(sums_ref[...], axis=0).reshape(NUM_CLASSES, D)
    counts = jnp.sum(cnts_ref[...], axis=0)[:, 0:1]
    protos = total / jnp.maximum(counts, 1.0)
    qe = jnp.dot(q_ref[...], w_ref[...].T)
    pn = jnp.sum(protos * protos, axis=1)[None, :]
    qp = lax.dot_general(qe, protos, (((1,), (1,)), ((), ())),
                         precision=lax.Precision.HIGHEST,
                         preferred_element_type=jnp.float32)
    scores = 2.0 * qp - pn
    m = jnp.max(scores, axis=1, keepdims=True)
    e = jnp.exp(scores - m)
    probs = e / jnp.sum(e, axis=1, keepdims=True)
    pred_ref[...] = jnp.argmax(scores, axis=1, keepdims=True).astype(jnp.int32)
    probs_ref[...] = probs


def _tc_tail(sums_p, cnts_p, query_data, W):
    nw = sums_p.shape[0]
    return pl.pallas_call(
        _tc_tail_body,
        out_shape=(
            jax.ShapeDtypeStruct((Q, 1), jnp.int32),
            jax.ShapeDtypeStruct((Q, NUM_CLASSES), jnp.float32),
        ),
    )(sums_p, cnts_p.reshape(nw, NUM_CLASSES, 16), query_data, W)


def kernel(support_data, query_data, W, labels):
    lab32 = labels.astype(jnp.int32)
    sums_parts, cnts_parts = [], []
    for k in range(NCHUNK):
        emb = _tc_embed(support_data[k * CHUNK:(k + 1) * CHUNK], W)
        s_p, c_p = _sc_segment_sum(emb, lab32[k * CHUNK:(k + 1) * CHUNK])
        sums_parts.append(s_p)
        cnts_parts.append(c_p)
    sums_p = jnp.concatenate(sums_parts, axis=0)
    cnts_p = jnp.concatenate(cnts_parts, axis=0)
    pred, probs = _tc_tail(sums_p, cnts_p, query_data, W)
    return pred.reshape((Q,)), probs
